# Initial kernel scaffold; baseline (speedup 1.0000x reference)
import jax
import jax.numpy as jnp
from jax.experimental import pallas as pl


def kernel(x, embed_table):
    raise NotImplementedError("write your pallas kernel here")


# speedup vs baseline: 2.8293x; 2.8293x over previous
import functools

import jax
import jax.numpy as jnp
from jax import lax
from jax.experimental import pallas as pl
from jax.experimental.pallas import tpu as pltpu
from jax.experimental.pallas import tpu_sc as plsc

_BINS = 1024
_WIDTH = 128
_NC = 2
_NS = 16
_NW = _NC * _NS
_LANES = 16
_C = 128


def _sc_body(x_hbm, table_hbm, out_hbm, x_v, idx_v, rows_v, sem):
    b_per_w = x_hbm.shape[0] // _NW
    n_chunk = b_per_w // _C
    wid = lax.axis_index("s") * _NC + lax.axis_index("c")
    base = wid * b_per_w

    pltpu.sync_copy(x_hbm.at[pl.ds(base, b_per_w)], x_v)

    def qbody(i, carry):
        xv = x_v[pl.ds(i * _LANES, _LANES)]
        q = (xv * float(_BINS)).astype(jnp.int32)
        idx_v[pl.ds(i * _LANES, _LANES)] = jnp.clip(q, 0, _BINS - 1)
        return carry

    lax.fori_loop(0, b_per_w // _LANES, qbody, 0)

    def cbody(c, carry):
        idx_slice = idx_v.at[pl.ds(c * _C, _C)]
        pltpu.async_copy(table_hbm.at[idx_slice], rows_v, sem).wait()
        pltpu.sync_copy(rows_v, out_hbm.at[pl.ds(base + c * _C, _C)])
        return carry

    lax.fori_loop(0, n_chunk, cbody, 0)


def kernel(x, embed_table):
    n, s = x.shape
    b = n * s
    b_per_w = b // _NW
    mesh = plsc.VectorSubcoreMesh(core_axis_name="c", subcore_axis_name="s")

    call = functools.partial(
        pl.kernel,
        mesh=mesh,
        out_type=jax.ShapeDtypeStruct((b, _WIDTH), jnp.float32),
        scratch_types=[
            pltpu.VMEM((b_per_w,), jnp.float32),
            pltpu.VMEM((b_per_w,), jnp.int32),
            pltpu.VMEM((_C, _WIDTH), jnp.float32),
            pltpu.SemaphoreType.DMA,
        ],
    )(_sc_body)

    out = call(x.reshape(b), embed_table)
    return out.reshape(n, s, _WIDTH)


# speedup vs baseline: 2.9335x; 1.0368x over previous
import functools

import jax
import jax.numpy as jnp
from jax import lax
from jax.experimental import pallas as pl
from jax.experimental.pallas import tpu as pltpu
from jax.experimental.pallas import tpu_sc as plsc

_BINS = 1024
_WIDTH = 128
_NC = 2
_NS = 16
_NW = _NC * _NS
_LANES = 16
_C = 128
_NBUF = 4


def _sc_body(x_hbm, table_hbm, out_hbm, x_v, idx_v, *bufs):
    rows = bufs[:_NBUF]
    gsem = bufs[_NBUF:2 * _NBUF]
    ssem = bufs[2 * _NBUF:3 * _NBUF]
    b_per_w = x_hbm.shape[0] // _NW
    n_chunk = b_per_w // _C
    wid = lax.axis_index("s") * _NC + lax.axis_index("c")
    base = wid * b_per_w

    pltpu.sync_copy(x_hbm.at[pl.ds(base, b_per_w)], x_v)

    def quantize_chunk(c):
        for i in range(_C // _LANES):
            off = c * _C + i * _LANES
            xv = x_v[pl.ds(off, _LANES)]
            q = (xv * float(_BINS)).astype(jnp.int32)
            idx_v[pl.ds(off, _LANES)] = jnp.clip(q, 0, _BINS - 1)

    def gather_start(c, b):
        pltpu.async_copy(
            table_hbm.at[idx_v.at[pl.ds(c * _C, _C)]], rows[b], gsem[b])

    def gather_wait(c, b):
        pltpu.make_async_copy(
            table_hbm.at[idx_v.at[pl.ds(c * _C, _C)]], rows[b], gsem[b]).wait()

    def scatter_start(c, b):
        pltpu.async_copy(
            rows[b], out_hbm.at[pl.ds(base + c * _C, _C)], ssem[b])

    def scatter_wait(c, b):
        pltpu.make_async_copy(
            rows[b], out_hbm.at[pl.ds(base + c * _C, _C)], ssem[b]).wait()

    for b in range(_NBUF):
        quantize_chunk(b)
        gather_start(b, b)

    @pl.loop(0, n_chunk - _NBUF, step=_NBUF)
    def _main(g):
        for b in range(_NBUF):
            c = g + b
            quantize_chunk(c + _NBUF)
            gather_wait(c, b)
            scatter_start(c, b)
            scatter_wait(c, b)
            gather_start(c + _NBUF, b)

    for b in range(_NBUF):
        c = n_chunk - _NBUF + b
        gather_wait(c, b)
        scatter_start(c, b)
    for b in range(_NBUF):
        scatter_wait(n_chunk - _NBUF + b, b)


def kernel(x, embed_table):
    n, s = x.shape
    b = n * s
    b_per_w = b // _NW
    mesh = plsc.VectorSubcoreMesh(core_axis_name="c", subcore_axis_name="s")

    call = functools.partial(
        pl.kernel,
        mesh=mesh,
        out_type=jax.ShapeDtypeStruct((b, _WIDTH), jnp.float32),
        scratch_types=(
            [pltpu.VMEM((b_per_w,), jnp.float32),
             pltpu.VMEM((b_per_w,), jnp.int32)]
            + [pltpu.VMEM((_C, _WIDTH), jnp.float32) for _ in range(_NBUF)]
            + [pltpu.SemaphoreType.DMA for _ in range(2 * _NBUF)]
        ),
    )(_sc_body)

    out = call(x.reshape(b), embed_table)
    return out.reshape(n, s, _WIDTH)


# speedup vs baseline: 3.9427x; 1.3440x over previous
import functools

import jax
import jax.numpy as jnp
from jax import lax
from jax.experimental import pallas as pl
from jax.experimental.pallas import tpu as pltpu
from jax.experimental.pallas import tpu_sc as plsc

_BINS = 1024
_WIDTH = 128
_NC = 2
_NS = 16
_NW = _NC * _NS
_LANES = 16
_C = 128
_NBUF = 4


def _sc_body(x_hbm, table_hbm, out_hbm, x_v, idx_v, table_sh, *bufs):
    rows = bufs[:_NBUF]
    gsem = bufs[_NBUF:2 * _NBUF]
    ssem = bufs[2 * _NBUF:3 * _NBUF]
    b_per_w = x_hbm.shape[0] // _NW
    n_chunk = b_per_w // _C
    sid = lax.axis_index("s")
    wid = sid * _NC + lax.axis_index("c")
    base = wid * b_per_w

    rows_per_sub = _BINS // _NS
    pltpu.sync_copy(table_hbm.at[pl.ds(sid * rows_per_sub, rows_per_sub)],
                    table_sh.at[pl.ds(sid * rows_per_sub, rows_per_sub)])

    @pl.when(sid == 0)
    def _last_row():
        pltpu.sync_copy(table_hbm.at[pl.ds(_BINS, 1)],
                        table_sh.at[pl.ds(_BINS, 1)])

    pltpu.sync_copy(x_hbm.at[pl.ds(base, b_per_w)], x_v)
    plsc.subcore_barrier()

    def quantize_chunk(c):
        for i in range(_C // _LANES):
            off = c * _C + i * _LANES
            xv = x_v[pl.ds(off, _LANES)]
            q = (xv * float(_BINS)).astype(jnp.int32)
            idx_v[pl.ds(off, _LANES)] = jnp.clip(q, 0, _BINS - 1)

    def gather_start(c, b):
        pltpu.async_copy(
            table_sh.at[idx_v.at[pl.ds(c * _C, _C)]], rows[b], gsem[b])

    def gather_wait(c, b):
        pltpu.make_async_copy(
            table_sh.at[idx_v.at[pl.ds(c * _C, _C)]], rows[b], gsem[b]).wait()

    def scatter_start(c, b):
        pltpu.async_copy(
            rows[b], out_hbm.at[pl.ds(base + c * _C, _C)], ssem[b])

    def scatter_wait(c, b):
        pltpu.make_async_copy(
            rows[b], out_hbm.at[pl.ds(base + c * _C, _C)], ssem[b]).wait()

    for b in range(_NBUF):
        quantize_chunk(b)
        gather_start(b, b)

    @pl.loop(0, n_chunk - _NBUF, step=_NBUF)
    def _main(g):
        for b in range(_NBUF):
            c = g + b
            quantize_chunk(c + _NBUF)
            gather_wait(c, b)
            scatter_start(c, b)
            scatter_wait(c, b)
            gather_start(c + _NBUF, b)

    for b in range(_NBUF):
        c = n_chunk - _NBUF + b
        gather_wait(c, b)
        scatter_start(c, b)
    for b in range(_NBUF):
        scatter_wait(n_chunk - _NBUF + b, b)


def kernel(x, embed_table):
    n, s = x.shape
    b = n * s
    b_per_w = b // _NW
    mesh = plsc.VectorSubcoreMesh(core_axis_name="c", subcore_axis_name="s")

    call = functools.partial(
        pl.kernel,
        mesh=mesh,
        out_type=jax.ShapeDtypeStruct((b, _WIDTH), jnp.float32),
        scratch_types=(
            [pltpu.VMEM((b_per_w,), jnp.float32),
             pltpu.VMEM((b_per_w,), jnp.int32),
             pltpu.VMEM_SHARED((_BINS + 1, _WIDTH), jnp.float32)]
            + [pltpu.VMEM((_C, _WIDTH), jnp.float32) for _ in range(_NBUF)]
            + [pltpu.SemaphoreType.DMA for _ in range(2 * _NBUF)]
        ),
    )(_sc_body)

    out = call(x.reshape(b), embed_table)
    return out.reshape(n, s, _WIDTH)


# speedup vs baseline: 7.9603x; 2.0190x over previous
import functools

import jax
import jax.numpy as jnp
from jax import lax
from jax.experimental import pallas as pl
from jax.experimental.pallas import tpu as pltpu
from jax.experimental.pallas import tpu_sc as plsc

_BINS = 1024
_WIDTH = 128
_NC = 2
_NS = 16
_NW = _NC * _NS
_LANES = 16
_SEQ = 50
_RPC = 4
_LPC = _RPC * _SEQ
_NBUF = 2
_GSPLIT = ((0, 128), (128, _LPC - 128))


def _sc_body(x_hbm, table_hbm, out_hbm, x_v, idx_v, table_sh, *bufs):
    rows = bufs[:_NBUF]
    gsem = bufs[_NBUF:2 * _NBUF]
    ssem = bufs[2 * _NBUF:3 * _NBUF]
    b_per_w = x_hbm.shape[0] // _NW
    n_chunk = b_per_w // _LPC
    rows_per_w = b_per_w // _SEQ
    sid = lax.axis_index("s")
    wid = sid * _NC + lax.axis_index("c")
    base = wid * b_per_w
    nrow_base = wid * rows_per_w

    rows_per_sub = _BINS // _NS
    pltpu.sync_copy(table_hbm.at[pl.ds(sid * rows_per_sub, rows_per_sub)],
                    table_sh.at[pl.ds(sid * rows_per_sub, rows_per_sub)])

    @pl.when(sid == 0)
    def _last_row():
        pltpu.sync_copy(table_hbm.at[pl.ds(_BINS, 1)],
                        table_sh.at[pl.ds(_BINS, 1)])

    pltpu.sync_copy(x_hbm.at[pl.ds(base, b_per_w)], x_v)

    def qbody(i, carry):
        xv = x_v[pl.ds(i * _LANES, _LANES)]
        q = (xv * float(_BINS)).astype(jnp.int32)
        idx_v[pl.ds(i * _LANES, _LANES)] = jnp.clip(q, 0, _BINS - 1)
        return carry

    lax.fori_loop(0, b_per_w // _LANES, qbody, 0)

    plsc.subcore_barrier()

    def gather_start(c, b):
        for o, cnt in _GSPLIT:
            pltpu.async_copy(table_sh.at[idx_v.at[pl.ds(c * _LPC + o, cnt)]],
                             rows[b].at[pl.ds(o, cnt)], gsem[b])

    def gather_wait(c, b):
        for o, cnt in _GSPLIT:
            pltpu.make_async_copy(
                table_sh.at[idx_v.at[pl.ds(c * _LPC + o, cnt)]],
                rows[b].at[pl.ds(o, cnt)], gsem[b]).wait()

    def scatter_start(c, b):
        for j in range(_RPC):
            pltpu.async_copy(rows[b].at[pl.ds(_SEQ * j, _SEQ)],
                             out_hbm.at[nrow_base + c * _RPC + j], ssem[b])

    def scatter_wait(c, b):
        for j in range(_RPC):
            pltpu.make_async_copy(rows[b].at[pl.ds(_SEQ * j, _SEQ)],
                                  out_hbm.at[nrow_base + c * _RPC + j],
                                  ssem[b]).wait()

    for b in range(_NBUF):
        gather_start(b, b)

    @pl.loop(0, n_chunk - _NBUF, step=_NBUF)
    def _main(g):
        for b in range(_NBUF):
            c = g + b
            gather_wait(c, b)
            scatter_start(c, b)
            scatter_wait(c, b)
            gather_start(c + _NBUF, b)

    for b in range(_NBUF):
        c = n_chunk - _NBUF + b
        gather_wait(c, b)
        scatter_start(c, b)
    for b in range(_NBUF):
        scatter_wait(n_chunk - _NBUF + b, b)


def kernel(x, embed_table):
    n, s = x.shape
    b = n * s
    b_per_w = b // _NW
    mesh = plsc.VectorSubcoreMesh(core_axis_name="c", subcore_axis_name="s")

    call = functools.partial(
        pl.kernel,
        mesh=mesh,
        out_type=jax.ShapeDtypeStruct((n, s, _WIDTH), jnp.float32),
        compiler_params=pltpu.CompilerParams(use_tc_tiling_on_sc=True),
        scratch_types=(
            [pltpu.VMEM((b_per_w,), jnp.float32),
             pltpu.VMEM((b_per_w,), jnp.int32),
             pltpu.VMEM_SHARED((_BINS + 1, _WIDTH), jnp.float32)]
            + [pltpu.VMEM((_LPC, _WIDTH), jnp.float32) for _ in range(_NBUF)]
            + [pltpu.SemaphoreType.DMA for _ in range(2 * _NBUF)]
        ),
    )(_sc_body)

    return call(x.reshape(b), embed_table)
